# baseline (speedup 1.0000x reference)
import functools

import jax
import jax.numpy as jnp
from jax import lax
from jax.experimental import pallas as pl
from jax.experimental.pallas import tpu as pltpu
from jax.experimental.pallas import tpu_sc as plsc

S = 7
P = S * S
PTS_PAD = 56
TAPS = 4 * P
TAPS_PAD = 208
H = W = 64
C = 128
FATC = 4 * C
FATW = FATC // 2
XY = H + 1
BROWS = XY * XY
NB = 2
NBOX = 1000
NBOXES = NB * NBOX
NBOXES_PAD = 2048
FROWS = NB * BROWS
OUT_ROW = C * P

NUM_TILES = 32
BPT = NBOXES_PAD // NUM_TILES
LANES = 16
CCHUNKS = C // LANES


def _sample_coords(b4, shape):
    f32 = jnp.float32
    cx = b4[..., 0:1]
    cy = b4[..., 1:2]
    bw = b4[..., 2:3]
    bh = b4[..., 3:4]
    x1 = (cx - bw * 0.5) * W
    y1 = (cy - bh * 0.5) * H
    x2 = (cx + bw * 0.5) * W
    y2 = (cy + bh * 0.5) * H
    step_x = (x2 - x1) / S
    step_y = (y2 - y1) / S
    return x1, y1, x2, y2, step_x, step_y


def _point_xy(ij, x1, y1, step_x, step_y):
    f32 = jnp.float32
    ijf = ij.astype(f32)
    i_f = jnp.floor(ijf / 7.0)
    j_f = ijf - i_f * 7.0
    px = x1 + (j_f + 0.5) * step_x
    py = y1 + (i_f + 0.5) * step_y
    gx = jnp.clip(px / W * 2.0 - 1.0, -1.0, 1.0)
    gy = jnp.clip(py / H * 2.0 - 1.0, -1.0, 1.0)
    ix = ((gx + 1.0) * W - 1.0) * 0.5
    iy = ((gy + 1.0) * H - 1.0) * 0.5
    return ix, iy


def _prep_body(boxes_ref, idx_ref, w_ref):
    b4 = boxes_ref[...]
    f32 = jnp.float32
    x1, y1, x2, y2, step_x, step_y = _sample_coords(b4, None)

    ishape = idx_ref.shape
    p = lax.broadcasted_iota(jnp.int32, ishape, 2)
    bidx = lax.broadcasted_iota(jnp.int32, ishape, 0)
    ij = jnp.minimum(p, P - 1)
    ix, iy = _point_xy(ij, x1, y1, step_x, step_y)
    x0 = jnp.floor(ix)
    y0 = jnp.floor(iy)
    idx_ref[...] = (bidx * BROWS
                    + (y0.astype(jnp.int32) + 1) * XY
                    + (x0.astype(jnp.int32) + 1))

    wshape = w_ref.shape
    pt = lax.broadcasted_iota(jnp.int32, wshape, 2)
    ij2 = lax.shift_right_logical(pt, 2)
    tt = jnp.bitwise_and(pt, 3)
    ix2, iy2 = _point_xy(ij2, x1, y1, step_x, step_y)
    x0b = jnp.floor(ix2)
    y0b = jnp.floor(iy2)
    fx = jnp.bitwise_and(tt, 1).astype(f32)
    fy = lax.shift_right_logical(tt, 1).astype(f32)
    xt = x0b + fx
    yt = y0b + fy
    wx1 = ix2 - x0b
    wy1 = iy2 - y0b
    wx = fx * wx1 + (1.0 - fx) * (1.0 - wx1)
    wy = fy * wy1 + (1.0 - fy) * (1.0 - wy1)
    valid = ((xt >= 0) & (xt <= W - 1) & (yt >= 0) & (yt <= H - 1))
    valid_box = (x2 > x1) & (y2 > y1)
    wt = (wy * wx) * valid.astype(f32) * valid_box.astype(f32)
    w_ref[...] = wt * (pt < TAPS).astype(f32)


_NBLK = 200


def _prep(boxes):
    grid = NBOX // _NBLK
    return pl.pallas_call(
        _prep_body,
        grid=(grid,),
        in_specs=[pl.BlockSpec((NB, _NBLK, 4), lambda i: (0, i, 0))],
        out_specs=(
            pl.BlockSpec((NB, _NBLK, PTS_PAD), lambda i: (0, i, 0)),
            pl.BlockSpec((NB, _NBLK, TAPS_PAD), lambda i: (0, i, 0)),
        ),
        out_shape=(
            jax.ShapeDtypeStruct((NB, NBOX, PTS_PAD), jnp.int32),
            jax.ShapeDtypeStruct((NB, NBOX, TAPS_PAD), jnp.float32),
        ),
    )(boxes)


def _sc_roi_kernel(table_hbm, idx_hbm, w_hbm, out_hbm,
                   idx_all, w_all, rows_v, out_v, sem):
    wid = lax.axis_index("s") * 2 + lax.axis_index("c")
    lane = jnp.arange(LANES, dtype=jnp.int32)
    zeros = jnp.zeros((LANES,), jnp.int32)
    lane2_p = (lane * 2) * P
    himask = jnp.full((LANES,), -65536, jnp.int32)

    pltpu.sync_copy(idx_hbm.at[pl.ds(wid * BPT, BPT)], idx_all)
    pltpu.sync_copy(w_hbm.at[pl.ds(wid * BPT, BPT)], w_all)

    def box_body(k, carry):
        t = wid * BPT + k
        pltpu.async_copy(table_hbm.at[idx_all.at[k]], rows_v, sem).wait()

        def pt(ij, c):
            tap0 = ij * 4
            wv = [plsc.load_gather(w_all, [zeros + k, zeros + (tap0 + tt)])
                  for tt in range(4)]
            for q in range(4):
                acc_e = None
                acc_o = None
                for tt in range(4):
                    v = rows_v[ij, pl.ds(tt * 64 + q * LANES, LANES)]
                    ev = plsc.bitcast(v << 16, jnp.float32)
                    ov = plsc.bitcast(v & himask, jnp.float32)
                    if acc_e is None:
                        acc_e = ev * wv[tt]
                        acc_o = ov * wv[tt]
                    else:
                        acc_e = acc_e + ev * wv[tt]
                        acc_o = acc_o + ov * wv[tt]
                st_e = lane2_p + ((32 * q) * P + ij)
                plsc.store_scatter(out_v, [st_e], acc_e)
                plsc.store_scatter(out_v, [st_e + P], acc_o)
            return c

        lax.fori_loop(0, P, pt, 0)

        @pl.when(t < NBOXES)
        def _():
            pltpu.sync_copy(out_v, out_hbm.at[t])

        return carry

    lax.fori_loop(0, BPT, box_body, 0)


@functools.cache
def _sc_roi():
    return pl.kernel(
        _sc_roi_kernel,
        mesh=plsc.VectorSubcoreMesh(core_axis_name="c", subcore_axis_name="s"),
        compiler_params=pltpu.CompilerParams(
            needs_layout_passes=False, use_tc_tiling_on_sc=False),
        out_type=jax.ShapeDtypeStruct((NBOXES, OUT_ROW), jnp.float32),
        scratch_types=[
            pltpu.VMEM((BPT, PTS_PAD), jnp.int32),
            pltpu.VMEM((BPT, TAPS_PAD), jnp.float32),
            pltpu.VMEM((PTS_PAD, FATW), jnp.int32),
            pltpu.VMEM((OUT_ROW,), jnp.float32),
            pltpu.SemaphoreType.DMA,
        ],
    )


def _fat_table(features):
    ft = jnp.transpose(features, (0, 2, 3, 1))
    pad = jnp.pad(ft, ((0, 0), (1, 1), (1, 1), (0, 0)))
    quads = [pad[:, dy:dy + XY, dx:dx + XY, :]
             for dy, dx in ((0, 0), (0, 1), (1, 0), (1, 1))]
    fat = jnp.concatenate(quads, axis=-1).reshape(FROWS, FATC)
    return lax.bitcast_convert_type(
        fat.astype(jnp.bfloat16).reshape(FROWS, FATW, 2), jnp.int32)


def kernel(features, boxes):
    table = _fat_table(features)
    idx3, w3 = _prep(boxes)
    padn = NBOXES_PAD - NBOXES
    idx2 = jnp.concatenate(
        [idx3.reshape(NBOXES, PTS_PAD), jnp.zeros((padn, PTS_PAD), jnp.int32)])
    w2 = jnp.concatenate(
        [w3.reshape(NBOXES, TAPS_PAD), jnp.zeros((padn, TAPS_PAD), jnp.float32)])
    out = _sc_roi()(table, idx2, w2)
    return out.reshape(NB, NBOX, C, S, S)


# speedup vs baseline: 1.1378x; 1.1378x over previous
import functools

import jax
import jax.numpy as jnp
from jax import lax
from jax.experimental import pallas as pl
from jax.experimental.pallas import tpu as pltpu
from jax.experimental.pallas import tpu_sc as plsc

S = 7
P = S * S
PTS_PAD = 56
TAPS = 4 * P
TAPS_PAD = 208
H = W = 64
C = 128
FATC = 4 * C
FATW = FATC // 2
XY = H + 1
BROWS = XY * XY
NB = 2
NBOX = 1000
NBOXES = NB * NBOX
NBOXES_PAD = 2048
FROWS = NB * BROWS
OUT_ROW = C * P

NUM_TILES = 32
BPT = NBOXES_PAD // NUM_TILES
LANES = 16
CCHUNKS = C // LANES


def _sample_coords(b4, shape):
    f32 = jnp.float32
    cx = b4[..., 0:1]
    cy = b4[..., 1:2]
    bw = b4[..., 2:3]
    bh = b4[..., 3:4]
    x1 = (cx - bw * 0.5) * W
    y1 = (cy - bh * 0.5) * H
    x2 = (cx + bw * 0.5) * W
    y2 = (cy + bh * 0.5) * H
    step_x = (x2 - x1) / S
    step_y = (y2 - y1) / S
    return x1, y1, x2, y2, step_x, step_y


def _point_xy(ij, x1, y1, step_x, step_y):
    f32 = jnp.float32
    ijf = ij.astype(f32)
    i_f = jnp.floor(ijf / 7.0)
    j_f = ijf - i_f * 7.0
    px = x1 + (j_f + 0.5) * step_x
    py = y1 + (i_f + 0.5) * step_y
    gx = jnp.clip(px / W * 2.0 - 1.0, -1.0, 1.0)
    gy = jnp.clip(py / H * 2.0 - 1.0, -1.0, 1.0)
    ix = ((gx + 1.0) * W - 1.0) * 0.5
    iy = ((gy + 1.0) * H - 1.0) * 0.5
    return ix, iy


def _prep_body(boxes_ref, idx_ref, w_ref):
    b4 = boxes_ref[...]
    f32 = jnp.float32
    x1, y1, x2, y2, step_x, step_y = _sample_coords(b4, None)

    ishape = idx_ref.shape
    p = lax.broadcasted_iota(jnp.int32, ishape, 2)
    bidx = lax.broadcasted_iota(jnp.int32, ishape, 0)
    ij = jnp.minimum(p, P - 1)
    ix, iy = _point_xy(ij, x1, y1, step_x, step_y)
    x0 = jnp.floor(ix)
    y0 = jnp.floor(iy)
    idx_ref[...] = (bidx * BROWS
                    + (y0.astype(jnp.int32) + 1) * XY
                    + (x0.astype(jnp.int32) + 1))

    wshape = w_ref.shape
    pt = lax.broadcasted_iota(jnp.int32, wshape, 2)
    ij2 = lax.shift_right_logical(pt, 2)
    tt = jnp.bitwise_and(pt, 3)
    ix2, iy2 = _point_xy(ij2, x1, y1, step_x, step_y)
    x0b = jnp.floor(ix2)
    y0b = jnp.floor(iy2)
    fx = jnp.bitwise_and(tt, 1).astype(f32)
    fy = lax.shift_right_logical(tt, 1).astype(f32)
    xt = x0b + fx
    yt = y0b + fy
    wx1 = ix2 - x0b
    wy1 = iy2 - y0b
    wx = fx * wx1 + (1.0 - fx) * (1.0 - wx1)
    wy = fy * wy1 + (1.0 - fy) * (1.0 - wy1)
    valid = ((xt >= 0) & (xt <= W - 1) & (yt >= 0) & (yt <= H - 1))
    valid_box = (x2 > x1) & (y2 > y1)
    wt = (wy * wx) * valid.astype(f32) * valid_box.astype(f32)
    w_ref[...] = wt * (pt < TAPS).astype(f32)


_NBLK = 200


def _prep(boxes):
    grid = NBOX // _NBLK
    return pl.pallas_call(
        _prep_body,
        grid=(grid,),
        in_specs=[pl.BlockSpec((NB, _NBLK, 4), lambda i: (0, i, 0))],
        out_specs=(
            pl.BlockSpec((NB, _NBLK, PTS_PAD), lambda i: (0, i, 0)),
            pl.BlockSpec((NB, _NBLK, TAPS_PAD), lambda i: (0, i, 0)),
        ),
        out_shape=(
            jax.ShapeDtypeStruct((NB, NBOX, PTS_PAD), jnp.int32),
            jax.ShapeDtypeStruct((NB, NBOX, TAPS_PAD), jnp.float32),
        ),
    )(boxes)


def _sc_roi_kernel(table_hbm, idx_hbm, w_hbm, out_hbm,
                   idx_all, w_all, rows0, rows1, out_v, sem0, sem1):
    wid = lax.axis_index("s") * 2 + lax.axis_index("c")
    lane = jnp.arange(LANES, dtype=jnp.int32)
    zeros = jnp.zeros((LANES,), jnp.int32)
    lane2_p = (lane * 2) * P
    himask = jnp.full((LANES,), -65536, jnp.int32)

    pltpu.sync_copy(idx_hbm.at[pl.ds(wid * BPT, BPT)], idx_all)
    pltpu.sync_copy(w_hbm.at[pl.ds(wid * BPT, BPT)], w_all)

    def make_pt(rows_v, k):
        def pt(ij, c):
            tap0 = ij * 4
            wv = [plsc.load_gather(w_all, [zeros + k, zeros + (tap0 + tt)])
                  for tt in range(4)]
            for q in range(4):
                acc_e = None
                acc_o = None
                for tt in range(4):
                    v = rows_v[ij, pl.ds(tt * 64 + q * LANES, LANES)]
                    ev = plsc.bitcast(v << 16, jnp.float32)
                    ov = plsc.bitcast(v & himask, jnp.float32)
                    if acc_e is None:
                        acc_e = ev * wv[tt]
                        acc_o = ov * wv[tt]
                    else:
                        acc_e = acc_e + ev * wv[tt]
                        acc_o = acc_o + ov * wv[tt]
                st_e = lane2_p + ((32 * q) * P + ij)
                plsc.store_scatter(out_v, [st_e], acc_e)
                plsc.store_scatter(out_v, [st_e + P], acc_o)
            return c

        return pt

    def compute_and_store(rows_v, k):
        t = wid * BPT + k
        lax.fori_loop(0, P, make_pt(rows_v, k), 0)

        @pl.when(t < NBOXES)
        def _():
            pltpu.sync_copy(out_v, out_hbm.at[t])

    def fetch(rows_v, sem, k):
        pltpu.async_copy(table_hbm.at[idx_all.at[k]], rows_v, sem)

    def wait(rows_v, sem, k):
        pltpu.make_async_copy(table_hbm.at[idx_all.at[k]], rows_v, sem).wait()

    def pair_body(p, carry):
        k0 = p * 2
        wait(rows0, sem0, k0)
        fetch(rows1, sem1, k0 + 1)
        compute_and_store(rows0, k0)
        wait(rows1, sem1, k0 + 1)

        @pl.when(p < BPT // 2 - 1)
        def _():
            fetch(rows0, sem0, k0 + 2)

        compute_and_store(rows1, k0 + 1)
        return carry

    fetch(rows0, sem0, 0)
    lax.fori_loop(0, BPT // 2, pair_body, 0)


@functools.cache
def _sc_roi():
    return pl.kernel(
        _sc_roi_kernel,
        mesh=plsc.VectorSubcoreMesh(core_axis_name="c", subcore_axis_name="s"),
        compiler_params=pltpu.CompilerParams(
            needs_layout_passes=False, use_tc_tiling_on_sc=False),
        out_type=jax.ShapeDtypeStruct((NBOXES, OUT_ROW), jnp.float32),
        scratch_types=[
            pltpu.VMEM((BPT, PTS_PAD), jnp.int32),
            pltpu.VMEM((BPT, TAPS_PAD), jnp.float32),
            pltpu.VMEM((PTS_PAD, FATW), jnp.int32),
            pltpu.VMEM((PTS_PAD, FATW), jnp.int32),
            pltpu.VMEM((OUT_ROW,), jnp.float32),
            pltpu.SemaphoreType.DMA,
            pltpu.SemaphoreType.DMA,
        ],
    )


def _fat_table(features):
    ft = jnp.transpose(features, (0, 2, 3, 1))
    pad = jnp.pad(ft, ((0, 0), (1, 1), (1, 1), (0, 0)))
    quads = [pad[:, dy:dy + XY, dx:dx + XY, :]
             for dy, dx in ((0, 0), (0, 1), (1, 0), (1, 1))]
    fat = jnp.concatenate(quads, axis=-1).reshape(FROWS, FATC)
    return lax.bitcast_convert_type(
        fat.astype(jnp.bfloat16).reshape(FROWS, FATW, 2), jnp.int32)


def kernel(features, boxes):
    table = _fat_table(features)
    idx3, w3 = _prep(boxes)
    padn = NBOXES_PAD - NBOXES
    idx2 = jnp.concatenate(
        [idx3.reshape(NBOXES, PTS_PAD), jnp.zeros((padn, PTS_PAD), jnp.int32)])
    w2 = jnp.concatenate(
        [w3.reshape(NBOXES, TAPS_PAD), jnp.zeros((padn, TAPS_PAD), jnp.float32)])
    out = _sc_roi()(table, idx2, w2)
    return out.reshape(NB, NBOX, C, S, S)


# speedup vs baseline: 1.1435x; 1.0050x over previous
import functools

import jax
import jax.numpy as jnp
from jax import lax
from jax.experimental import pallas as pl
from jax.experimental.pallas import tpu as pltpu
from jax.experimental.pallas import tpu_sc as plsc

S = 7
P = S * S
PTS_PAD = 56
TAPS = 4 * P
TAPS_PAD = 208
H = W = 64
C = 128
FATC = 4 * C
FATW = FATC // 2
XY = H + 1
BROWS = XY * XY
NB = 2
NBOX = 1000
NBOXES = NB * NBOX
NBOXES_PAD = 2048
FROWS = NB * BROWS
OUT_ROW = C * P

NUM_TILES = 32
BPT = NBOXES_PAD // NUM_TILES
LANES = 16
CCHUNKS = C // LANES


def _sample_coords(b4, shape):
    f32 = jnp.float32
    cx = b4[..., 0:1]
    cy = b4[..., 1:2]
    bw = b4[..., 2:3]
    bh = b4[..., 3:4]
    x1 = (cx - bw * 0.5) * W
    y1 = (cy - bh * 0.5) * H
    x2 = (cx + bw * 0.5) * W
    y2 = (cy + bh * 0.5) * H
    step_x = (x2 - x1) / S
    step_y = (y2 - y1) / S
    return x1, y1, x2, y2, step_x, step_y


def _point_xy(ij, x1, y1, step_x, step_y):
    f32 = jnp.float32
    ijf = ij.astype(f32)
    i_f = jnp.floor(ijf / 7.0)
    j_f = ijf - i_f * 7.0
    px = x1 + (j_f + 0.5) * step_x
    py = y1 + (i_f + 0.5) * step_y
    gx = jnp.clip(px / W * 2.0 - 1.0, -1.0, 1.0)
    gy = jnp.clip(py / H * 2.0 - 1.0, -1.0, 1.0)
    ix = ((gx + 1.0) * W - 1.0) * 0.5
    iy = ((gy + 1.0) * H - 1.0) * 0.5
    return ix, iy


def _prep_body(boxes_ref, idx_ref, w_ref):
    b4 = boxes_ref[...]
    f32 = jnp.float32
    x1, y1, x2, y2, step_x, step_y = _sample_coords(b4, None)

    ishape = idx_ref.shape
    p = lax.broadcasted_iota(jnp.int32, ishape, 2)
    bidx = lax.broadcasted_iota(jnp.int32, ishape, 0)
    ij = jnp.minimum(p, P - 1)
    ix, iy = _point_xy(ij, x1, y1, step_x, step_y)
    x0 = jnp.floor(ix)
    y0 = jnp.floor(iy)
    idx_ref[...] = (bidx * BROWS
                    + (y0.astype(jnp.int32) + 1) * XY
                    + (x0.astype(jnp.int32) + 1))

    wshape = w_ref.shape
    pt = lax.broadcasted_iota(jnp.int32, wshape, 2)
    ij2 = lax.shift_right_logical(pt, 2)
    tt = jnp.bitwise_and(pt, 3)
    ix2, iy2 = _point_xy(ij2, x1, y1, step_x, step_y)
    x0b = jnp.floor(ix2)
    y0b = jnp.floor(iy2)
    fx = jnp.bitwise_and(tt, 1).astype(f32)
    fy = lax.shift_right_logical(tt, 1).astype(f32)
    xt = x0b + fx
    yt = y0b + fy
    wx1 = ix2 - x0b
    wy1 = iy2 - y0b
    wx = fx * wx1 + (1.0 - fx) * (1.0 - wx1)
    wy = fy * wy1 + (1.0 - fy) * (1.0 - wy1)
    valid = ((xt >= 0) & (xt <= W - 1) & (yt >= 0) & (yt <= H - 1))
    valid_box = (x2 > x1) & (y2 > y1)
    wt = (wy * wx) * valid.astype(f32) * valid_box.astype(f32)
    w_ref[...] = wt * (pt < TAPS).astype(f32)


_NBLK = 200


def _prep(boxes):
    grid = NBOX // _NBLK
    return pl.pallas_call(
        _prep_body,
        grid=(grid,),
        in_specs=[pl.BlockSpec((NB, _NBLK, 4), lambda i: (0, i, 0))],
        out_specs=(
            pl.BlockSpec((NB, _NBLK, PTS_PAD), lambda i: (0, i, 0)),
            pl.BlockSpec((NB, _NBLK, TAPS_PAD), lambda i: (0, i, 0)),
        ),
        out_shape=(
            jax.ShapeDtypeStruct((NB, NBOX, PTS_PAD), jnp.int32),
            jax.ShapeDtypeStruct((NB, NBOX, TAPS_PAD), jnp.float32),
        ),
    )(boxes)


def _sc_roi_kernel(table_hbm, idx_hbm, w_hbm, out_hbm,
                   idx_all, w_all, rows0, rows1, out0, out1,
                   sem0, sem1, osem0, osem1):
    wid = lax.axis_index("s") * 2 + lax.axis_index("c")
    lane = jnp.arange(LANES, dtype=jnp.int32)
    zeros = jnp.zeros((LANES,), jnp.int32)
    lane2_p = (lane * 2) * P
    himask = jnp.full((LANES,), -65536, jnp.int32)

    pltpu.sync_copy(idx_hbm.at[pl.ds(wid * BPT, BPT)], idx_all)
    pltpu.sync_copy(w_hbm.at[pl.ds(wid * BPT, BPT)], w_all)

    def make_pt(rows_v, out_v, k):
        def pt(ij, c):
            tap0 = ij * 4
            wv = [plsc.load_gather(w_all, [zeros + k, zeros + (tap0 + tt)])
                  for tt in range(4)]
            for q in range(4):
                acc_e = None
                acc_o = None
                for tt in range(4):
                    v = rows_v[ij, pl.ds(tt * 64 + q * LANES, LANES)]
                    ev = plsc.bitcast(v << 16, jnp.float32)
                    ov = plsc.bitcast(v & himask, jnp.float32)
                    if acc_e is None:
                        acc_e = ev * wv[tt]
                        acc_o = ov * wv[tt]
                    else:
                        acc_e = acc_e + ev * wv[tt]
                        acc_o = acc_o + ov * wv[tt]
                st_e = lane2_p + ((32 * q) * P + ij)
                plsc.store_scatter(out_v, [st_e], acc_e)
                plsc.store_scatter(out_v, [st_e + P], acc_o)
            return c

        return pt

    def compute_and_store(rows_v, out_v, osem, k, p):
        t = wid * BPT + k
        t_prev = t - 2

        @pl.when((p > 0) & (t_prev < NBOXES))
        def _():
            pltpu.make_async_copy(out_v, out_hbm.at[t_prev], osem).wait()

        lax.fori_loop(0, P, make_pt(rows_v, out_v, k), 0)

        @pl.when(t < NBOXES)
        def _():
            pltpu.async_copy(out_v, out_hbm.at[t], osem)

    def fetch(rows_v, sem, k):
        pltpu.async_copy(table_hbm.at[idx_all.at[k]], rows_v, sem)

    def wait(rows_v, sem, k):
        pltpu.make_async_copy(table_hbm.at[idx_all.at[k]], rows_v, sem).wait()

    def pair_body(p, carry):
        k0 = p * 2
        wait(rows0, sem0, k0)
        fetch(rows1, sem1, k0 + 1)
        compute_and_store(rows0, out0, osem0, k0, p)
        wait(rows1, sem1, k0 + 1)

        @pl.when(p < BPT // 2 - 1)
        def _():
            fetch(rows0, sem0, k0 + 2)

        compute_and_store(rows1, out1, osem1, k0 + 1, p)
        return carry

    fetch(rows0, sem0, 0)
    lax.fori_loop(0, BPT // 2, pair_body, 0)

    for out_v, osem, kl in ((out0, osem0, BPT - 2), (out1, osem1, BPT - 1)):
        t_last = wid * BPT + kl

        @pl.when(t_last < NBOXES)
        def _(out_v=out_v, osem=osem, t_last=t_last):
            pltpu.make_async_copy(out_v, out_hbm.at[t_last], osem).wait()


@functools.cache
def _sc_roi():
    return pl.kernel(
        _sc_roi_kernel,
        mesh=plsc.VectorSubcoreMesh(core_axis_name="c", subcore_axis_name="s"),
        compiler_params=pltpu.CompilerParams(
            needs_layout_passes=False, use_tc_tiling_on_sc=False),
        out_type=jax.ShapeDtypeStruct((NBOXES, OUT_ROW), jnp.float32),
        scratch_types=[
            pltpu.VMEM((BPT, PTS_PAD), jnp.int32),
            pltpu.VMEM((BPT, TAPS_PAD), jnp.float32),
            pltpu.VMEM((PTS_PAD, FATW), jnp.int32),
            pltpu.VMEM((PTS_PAD, FATW), jnp.int32),
            pltpu.VMEM((OUT_ROW,), jnp.float32),
            pltpu.VMEM((OUT_ROW,), jnp.float32),
            pltpu.SemaphoreType.DMA,
            pltpu.SemaphoreType.DMA,
            pltpu.SemaphoreType.DMA,
            pltpu.SemaphoreType.DMA,
        ],
    )


def _fat_table(features):
    ft = jnp.transpose(features, (0, 2, 3, 1))
    pad = jnp.pad(ft, ((0, 0), (1, 1), (1, 1), (0, 0)))
    quads = [pad[:, dy:dy + XY, dx:dx + XY, :]
             for dy, dx in ((0, 0), (0, 1), (1, 0), (1, 1))]
    fat = jnp.concatenate(quads, axis=-1).reshape(FROWS, FATC)
    return lax.bitcast_convert_type(
        fat.astype(jnp.bfloat16).reshape(FROWS, FATW, 2), jnp.int32)


def kernel(features, boxes):
    table = _fat_table(features)
    idx3, w3 = _prep(boxes)
    padn = NBOXES_PAD - NBOXES
    idx2 = jnp.concatenate(
        [idx3.reshape(NBOXES, PTS_PAD), jnp.zeros((padn, PTS_PAD), jnp.int32)])
    w2 = jnp.concatenate(
        [w3.reshape(NBOXES, TAPS_PAD), jnp.zeros((padn, TAPS_PAD), jnp.float32)])
    out = _sc_roi()(table, idx2, w2)
    return out.reshape(NB, NBOX, C, S, S)


# speedup vs baseline: 1.1594x; 1.0139x over previous
import functools

import jax
import jax.numpy as jnp
from jax import lax
from jax.experimental import pallas as pl
from jax.experimental.pallas import tpu as pltpu
from jax.experimental.pallas import tpu_sc as plsc

S = 7
P = S * S
PTS_PAD = 56
TAPS = 4 * P
TAPS_PAD = 208
H = W = 64
C = 128
FATC = 4 * C
FATW = FATC // 2
XY = H + 1
BROWS = XY * XY
NB = 2
NBOX = 1000
NBOXES = NB * NBOX
NBOXES_PAD = 2048
FROWS = NB * BROWS
OUT_ROW = C * P

NUM_TILES = 32
BPT = NBOXES_PAD // NUM_TILES
LANES = 16
CCHUNKS = C // LANES


def _sample_coords(b4, shape):
    f32 = jnp.float32
    cx = b4[..., 0:1]
    cy = b4[..., 1:2]
    bw = b4[..., 2:3]
    bh = b4[..., 3:4]
    x1 = (cx - bw * 0.5) * W
    y1 = (cy - bh * 0.5) * H
    x2 = (cx + bw * 0.5) * W
    y2 = (cy + bh * 0.5) * H
    step_x = (x2 - x1) / S
    step_y = (y2 - y1) / S
    return x1, y1, x2, y2, step_x, step_y


def _point_xy(ij, x1, y1, step_x, step_y):
    f32 = jnp.float32
    ijf = ij.astype(f32)
    i_f = jnp.floor(ijf / 7.0)
    j_f = ijf - i_f * 7.0
    px = x1 + (j_f + 0.5) * step_x
    py = y1 + (i_f + 0.5) * step_y
    gx = jnp.clip(px / W * 2.0 - 1.0, -1.0, 1.0)
    gy = jnp.clip(py / H * 2.0 - 1.0, -1.0, 1.0)
    ix = ((gx + 1.0) * W - 1.0) * 0.5
    iy = ((gy + 1.0) * H - 1.0) * 0.5
    return ix, iy


def _prep_body(boxes_ref, idx_ref, w_ref):
    b4 = boxes_ref[...]
    f32 = jnp.float32
    x1, y1, x2, y2, step_x, step_y = _sample_coords(b4, None)

    ishape = idx_ref.shape
    p = lax.broadcasted_iota(jnp.int32, ishape, 2)
    bidx = lax.broadcasted_iota(jnp.int32, ishape, 0)
    ij = jnp.minimum(p, P - 1)
    ix, iy = _point_xy(ij, x1, y1, step_x, step_y)
    x0 = jnp.floor(ix)
    y0 = jnp.floor(iy)
    idx_ref[...] = (bidx * BROWS
                    + (y0.astype(jnp.int32) + 1) * XY
                    + (x0.astype(jnp.int32) + 1))

    wshape = w_ref.shape
    pt = lax.broadcasted_iota(jnp.int32, wshape, 2)
    ij2 = lax.shift_right_logical(pt, 2)
    tt = jnp.bitwise_and(pt, 3)
    ix2, iy2 = _point_xy(ij2, x1, y1, step_x, step_y)
    x0b = jnp.floor(ix2)
    y0b = jnp.floor(iy2)
    fx = jnp.bitwise_and(tt, 1).astype(f32)
    fy = lax.shift_right_logical(tt, 1).astype(f32)
    xt = x0b + fx
    yt = y0b + fy
    wx1 = ix2 - x0b
    wy1 = iy2 - y0b
    wx = fx * wx1 + (1.0 - fx) * (1.0 - wx1)
    wy = fy * wy1 + (1.0 - fy) * (1.0 - wy1)
    valid = ((xt >= 0) & (xt <= W - 1) & (yt >= 0) & (yt <= H - 1))
    valid_box = (x2 > x1) & (y2 > y1)
    wt = (wy * wx) * valid.astype(f32) * valid_box.astype(f32)
    w_ref[...] = wt * (pt < TAPS).astype(f32)


_NBLK = 200


def _prep(boxes):
    grid = NBOX // _NBLK
    return pl.pallas_call(
        _prep_body,
        grid=(grid,),
        in_specs=[pl.BlockSpec((NB, _NBLK, 4), lambda i: (0, i, 0))],
        out_specs=(
            pl.BlockSpec((NB, _NBLK, PTS_PAD), lambda i: (0, i, 0)),
            pl.BlockSpec((NB, _NBLK, TAPS_PAD), lambda i: (0, i, 0)),
        ),
        out_shape=(
            jax.ShapeDtypeStruct((NB, NBOX, PTS_PAD), jnp.int32),
            jax.ShapeDtypeStruct((NB, NBOX, TAPS_PAD), jnp.float32),
        ),
    )(boxes)


def _sc_roi_kernel(table_hbm, idx_hbm, w_hbm, out_hbm,
                   idx_all, w_all, rows0, rows1, out0, out1,
                   sem0, sem1, osem0, osem1):
    wid = lax.axis_index("s") * 2 + lax.axis_index("c")
    lane = jnp.arange(LANES, dtype=jnp.int32)
    zeros = jnp.zeros((LANES,), jnp.int32)
    lane2_p = (lane * 2) * P
    himask = jnp.full((LANES,), -65536, jnp.int32)

    pltpu.sync_copy(idx_hbm.at[pl.ds(wid * (BPT // 2), BPT // 2)], idx_all)
    pltpu.sync_copy(w_hbm.at[pl.ds(wid * BPT, BPT)], w_all)

    def make_pt(rows_v, out_v, b2, k):
        def pt(ij, c):
            tap0 = ij * 4
            wv = [plsc.load_gather(w_all, [zeros + k, zeros + (tap0 + tt)])
                  for tt in range(4)]
            box_sel = zeros + b2
            for q in range(4):
                acc_e = None
                acc_o = None
                for tt in range(4):
                    v = rows_v[b2 * PTS_PAD + ij,
                               pl.ds(tt * 64 + q * LANES, LANES)]
                    ev = plsc.bitcast(v << 16, jnp.float32)
                    ov = plsc.bitcast(v & himask, jnp.float32)
                    if acc_e is None:
                        acc_e = ev * wv[tt]
                        acc_o = ov * wv[tt]
                    else:
                        acc_e = acc_e + ev * wv[tt]
                        acc_o = acc_o + ov * wv[tt]
                st_e = lane2_p + ((32 * q) * P + ij)
                plsc.store_scatter(out_v, [box_sel, st_e], acc_e)
                plsc.store_scatter(out_v, [box_sel, st_e + P], acc_o)
            return c

        return pt

    M = 2

    def compute_and_store(rows_v, out_v, osem, kc, p):
        tb = wid * BPT + kc * M
        t_prev = tb - 2 * M

        @pl.when((p > 0) & (t_prev < NBOXES))
        def _():
            pltpu.make_async_copy(out_v, out_hbm.at[pl.ds(t_prev, M)],
                                  osem).wait()

        for b2 in range(M):
            lax.fori_loop(0, P, make_pt(rows_v, out_v, b2, kc * M + b2), 0)

        @pl.when(tb < NBOXES)
        def _():
            pltpu.async_copy(out_v, out_hbm.at[pl.ds(tb, M)], osem)

    def fetch(rows_v, sem, kc):
        pltpu.async_copy(table_hbm.at[idx_all.at[kc]], rows_v, sem)

    def wait(rows_v, sem, kc):
        pltpu.make_async_copy(table_hbm.at[idx_all.at[kc]],
                              rows_v, sem).wait()

    NCH = BPT // M

    def pair_body(p, carry):
        kc0 = p * 2
        wait(rows0, sem0, kc0)
        fetch(rows1, sem1, kc0 + 1)
        compute_and_store(rows0, out0, osem0, kc0, p)
        wait(rows1, sem1, kc0 + 1)

        @pl.when(p < NCH // 2 - 1)
        def _():
            fetch(rows0, sem0, kc0 + 2)

        compute_and_store(rows1, out1, osem1, kc0 + 1, p)
        return carry

    fetch(rows0, sem0, 0)
    lax.fori_loop(0, NCH // 2, pair_body, 0)

    for out_v, osem, kl in ((out0, osem0, NCH - 2), (out1, osem1, NCH - 1)):
        t_last = wid * BPT + kl * M

        @pl.when(t_last < NBOXES)
        def _(out_v=out_v, osem=osem, t_last=t_last):
            pltpu.make_async_copy(out_v, out_hbm.at[pl.ds(t_last, M)],
                                  osem).wait()


@functools.cache
def _sc_roi():
    return pl.kernel(
        _sc_roi_kernel,
        mesh=plsc.VectorSubcoreMesh(core_axis_name="c", subcore_axis_name="s"),
        compiler_params=pltpu.CompilerParams(
            needs_layout_passes=False, use_tc_tiling_on_sc=False),
        out_type=jax.ShapeDtypeStruct((NBOXES, OUT_ROW), jnp.float32),
        scratch_types=[
            pltpu.VMEM((BPT // 2, 2 * PTS_PAD), jnp.int32),
            pltpu.VMEM((BPT, TAPS_PAD), jnp.float32),
            pltpu.VMEM((2 * PTS_PAD, FATW), jnp.int32),
            pltpu.VMEM((2 * PTS_PAD, FATW), jnp.int32),
            pltpu.VMEM((2, OUT_ROW), jnp.float32),
            pltpu.VMEM((2, OUT_ROW), jnp.float32),
            pltpu.SemaphoreType.DMA,
            pltpu.SemaphoreType.DMA,
            pltpu.SemaphoreType.DMA,
            pltpu.SemaphoreType.DMA,
        ],
    )


def _fat_table(features):
    ft = jnp.transpose(features, (0, 2, 3, 1))
    pad = jnp.pad(ft, ((0, 0), (1, 1), (1, 1), (0, 0)))
    quads = [pad[:, dy:dy + XY, dx:dx + XY, :]
             for dy, dx in ((0, 0), (0, 1), (1, 0), (1, 1))]
    fat = jnp.concatenate(quads, axis=-1).reshape(FROWS, FATC)
    return lax.bitcast_convert_type(
        fat.astype(jnp.bfloat16).reshape(FROWS, FATW, 2), jnp.int32)


def kernel(features, boxes):
    table = _fat_table(features)
    idx3, w3 = _prep(boxes)
    padn = NBOXES_PAD - NBOXES
    idx2 = jnp.concatenate(
        [idx3.reshape(NBOXES, PTS_PAD),
         jnp.zeros((padn, PTS_PAD), jnp.int32)]).reshape(
             NBOXES_PAD // 2, 2 * PTS_PAD)
    w2 = jnp.concatenate(
        [w3.reshape(NBOXES, TAPS_PAD), jnp.zeros((padn, TAPS_PAD), jnp.float32)])
    out = _sc_roi()(table, idx2, w2)
    return out.reshape(NB, NBOX, C, S, S)


# speedup vs baseline: 1.2379x; 1.0677x over previous
import functools

import jax
import jax.numpy as jnp
from jax import lax
from jax.experimental import pallas as pl
from jax.experimental.pallas import tpu as pltpu
from jax.experimental.pallas import tpu_sc as plsc

S = 7
P = S * S
PTS_PAD = 56
TAPS = 4 * P
TAPS_PAD = 208
H = W = 64
C = 128
FATC = 4 * C
FATW = FATC // 2
XY = H + 1
BROWS = XY * XY
NB = 2
NBOX = 1000
NBOXES = NB * NBOX
NBOXES_PAD = 2048
FROWS = NB * BROWS
OUT_ROW = C * P

NUM_TILES = 32
BPT = NBOXES_PAD // NUM_TILES
LANES = 16
CCHUNKS = C // LANES


def _sample_coords(b4, shape):
    f32 = jnp.float32
    cx = b4[..., 0:1]
    cy = b4[..., 1:2]
    bw = b4[..., 2:3]
    bh = b4[..., 3:4]
    x1 = (cx - bw * 0.5) * W
    y1 = (cy - bh * 0.5) * H
    x2 = (cx + bw * 0.5) * W
    y2 = (cy + bh * 0.5) * H
    step_x = (x2 - x1) / S
    step_y = (y2 - y1) / S
    return x1, y1, x2, y2, step_x, step_y


def _point_xy(ij, x1, y1, step_x, step_y):
    f32 = jnp.float32
    ijf = ij.astype(f32)
    i_f = jnp.floor(ijf / 7.0)
    j_f = ijf - i_f * 7.0
    px = x1 + (j_f + 0.5) * step_x
    py = y1 + (i_f + 0.5) * step_y
    gx = jnp.clip(px / W * 2.0 - 1.0, -1.0, 1.0)
    gy = jnp.clip(py / H * 2.0 - 1.0, -1.0, 1.0)
    ix = ((gx + 1.0) * W - 1.0) * 0.5
    iy = ((gy + 1.0) * H - 1.0) * 0.5
    return ix, iy


def _prep_body(boxes_ref, idx_ref, w_ref):
    b4 = boxes_ref[...]
    f32 = jnp.float32
    x1, y1, x2, y2, step_x, step_y = _sample_coords(b4, None)

    ishape = idx_ref.shape
    p = lax.broadcasted_iota(jnp.int32, ishape, 2)
    bidx = lax.broadcasted_iota(jnp.int32, ishape, 0)
    ij = jnp.minimum(p, P - 1)
    ix, iy = _point_xy(ij, x1, y1, step_x, step_y)
    x0 = jnp.floor(ix)
    y0 = jnp.floor(iy)
    idx_ref[...] = (bidx * BROWS
                    + (y0.astype(jnp.int32) + 1) * XY
                    + (x0.astype(jnp.int32) + 1))

    wshape = w_ref.shape
    pt = lax.broadcasted_iota(jnp.int32, wshape, 2)
    ij2 = lax.shift_right_logical(pt, 2)
    tt = jnp.bitwise_and(pt, 3)
    ix2, iy2 = _point_xy(ij2, x1, y1, step_x, step_y)
    x0b = jnp.floor(ix2)
    y0b = jnp.floor(iy2)
    fx = jnp.bitwise_and(tt, 1).astype(f32)
    fy = lax.shift_right_logical(tt, 1).astype(f32)
    xt = x0b + fx
    yt = y0b + fy
    wx1 = ix2 - x0b
    wy1 = iy2 - y0b
    wx = fx * wx1 + (1.0 - fx) * (1.0 - wx1)
    wy = fy * wy1 + (1.0 - fy) * (1.0 - wy1)
    valid = ((xt >= 0) & (xt <= W - 1) & (yt >= 0) & (yt <= H - 1))
    valid_box = (x2 > x1) & (y2 > y1)
    wt = (wy * wx) * valid.astype(f32) * valid_box.astype(f32)
    w_ref[...] = wt * (pt < TAPS).astype(f32)


_NBLK = 200


def _prep(boxes):
    grid = NBOX // _NBLK
    return pl.pallas_call(
        _prep_body,
        grid=(grid,),
        in_specs=[pl.BlockSpec((NB, _NBLK, 4), lambda i: (0, i, 0))],
        out_specs=(
            pl.BlockSpec((NB, _NBLK, PTS_PAD), lambda i: (0, i, 0)),
            pl.BlockSpec((NB, _NBLK, TAPS_PAD), lambda i: (0, i, 0)),
        ),
        out_shape=(
            jax.ShapeDtypeStruct((NB, NBOX, PTS_PAD), jnp.int32),
            jax.ShapeDtypeStruct((NB, NBOX, TAPS_PAD), jnp.float32),
        ),
    )(boxes)


def _sc_roi_kernel(table_hbm, idx_hbm, w_hbm, out_hbm,
                   idx_all, w_all, rows0, rows1, out0, out1,
                   sem0, sem1, osem0, osem1):
    wid = lax.axis_index("s") * 2 + lax.axis_index("c")
    lane = jnp.arange(LANES, dtype=jnp.int32)
    zeros = jnp.zeros((LANES,), jnp.int32)
    lane2_p = (lane * 2) * P
    himask = jnp.full((LANES,), -65536, jnp.int32)

    pltpu.sync_copy(idx_hbm.at[pl.ds(wid * (BPT // 2), BPT // 2)], idx_all)
    pltpu.sync_copy(w_hbm.at[pl.ds(wid * BPT, BPT)], w_all)

    def make_pt(rows_v, out_v, b2, k):
        def pt(ij, c):
            tap0 = ij * 4
            wv = [plsc.load_gather(w_all, [zeros + k, zeros + (tap0 + tt)])
                  for tt in range(4)]
            box_sel = zeros + b2
            for q in range(4):
                acc_e = None
                acc_o = None
                for tt in range(4):
                    v = rows_v[b2 * PTS_PAD + ij,
                               pl.ds(tt * 64 + q * LANES, LANES)]
                    ev = plsc.bitcast(v << 16, jnp.float32)
                    ov = plsc.bitcast(v & himask, jnp.float32)
                    if acc_e is None:
                        acc_e = ev * wv[tt]
                        acc_o = ov * wv[tt]
                    else:
                        acc_e = acc_e + ev * wv[tt]
                        acc_o = acc_o + ov * wv[tt]
                st_e = lane2_p + ((32 * q) * P + ij)
                plsc.store_scatter(out_v, [box_sel, st_e], acc_e)
                plsc.store_scatter(out_v, [box_sel, st_e + P], acc_o)
            return c

        return pt

    M = 2

    def compute_and_store(rows_v, out_v, osem, kc, p):
        tb = wid * BPT + kc * M
        t_prev = tb - 2 * M

        @pl.when((p > 0) & (t_prev < NBOXES))
        def _():
            pltpu.make_async_copy(out_v, out_hbm.at[pl.ds(t_prev, M)],
                                  osem).wait()

        for b2 in range(M):
            lax.fori_loop(0, P, make_pt(rows_v, out_v, b2, kc * M + b2), 0)

        @pl.when(tb < NBOXES)
        def _():
            pltpu.async_copy(out_v, out_hbm.at[pl.ds(tb, M)], osem)

    def fetch(rows_v, sem, kc):
        pltpu.async_copy(table_hbm.at[idx_all.at[kc]], rows_v, sem)

    def wait(rows_v, sem, kc):
        pltpu.make_async_copy(table_hbm.at[idx_all.at[kc]],
                              rows_v, sem).wait()

    NCH = BPT // M

    def pair_body(p, carry):
        kc0 = p * 2
        wait(rows0, sem0, kc0)
        fetch(rows1, sem1, kc0 + 1)
        compute_and_store(rows0, out0, osem0, kc0, p)
        wait(rows1, sem1, kc0 + 1)

        @pl.when(p < NCH // 2 - 1)
        def _():
            fetch(rows0, sem0, kc0 + 2)

        compute_and_store(rows1, out1, osem1, kc0 + 1, p)
        return carry

    fetch(rows0, sem0, 0)
    lax.fori_loop(0, NCH // 2, pair_body, 0)

    for out_v, osem, kl in ((out0, osem0, NCH - 2), (out1, osem1, NCH - 1)):
        t_last = wid * BPT + kl * M

        @pl.when(t_last < NBOXES)
        def _(out_v=out_v, osem=osem, t_last=t_last):
            pltpu.make_async_copy(out_v, out_hbm.at[pl.ds(t_last, M)],
                                  osem).wait()


@functools.cache
def _sc_roi():
    return pl.kernel(
        _sc_roi_kernel,
        mesh=plsc.VectorSubcoreMesh(core_axis_name="c", subcore_axis_name="s"),
        compiler_params=pltpu.CompilerParams(needs_layout_passes=False),
        out_type=jax.ShapeDtypeStruct((NBOXES, OUT_ROW), jnp.float32),
        scratch_types=[
            pltpu.VMEM((BPT // 2, 2 * PTS_PAD), jnp.int32),
            pltpu.VMEM((BPT, TAPS_PAD), jnp.float32),
            pltpu.VMEM((2 * PTS_PAD, FATW), jnp.int32),
            pltpu.VMEM((2 * PTS_PAD, FATW), jnp.int32),
            pltpu.VMEM((2, OUT_ROW), jnp.float32),
            pltpu.VMEM((2, OUT_ROW), jnp.float32),
            pltpu.SemaphoreType.DMA,
            pltpu.SemaphoreType.DMA,
            pltpu.SemaphoreType.DMA,
            pltpu.SemaphoreType.DMA,
        ],
    )


def _fat_table(features):
    ft = jnp.transpose(features, (0, 2, 3, 1))
    pad = jnp.pad(ft, ((0, 0), (1, 1), (1, 1), (0, 0)))
    quads = [pad[:, dy:dy + XY, dx:dx + XY, :]
             for dy, dx in ((0, 0), (0, 1), (1, 0), (1, 1))]
    fat = jnp.concatenate(quads, axis=-1).reshape(FROWS, FATC)
    return lax.bitcast_convert_type(
        fat.astype(jnp.bfloat16).reshape(FROWS, FATW, 2), jnp.int32)


def kernel(features, boxes):
    table = _fat_table(features)
    idx3, w3 = _prep(boxes)
    padn = NBOXES_PAD - NBOXES
    idx2 = jnp.concatenate(
        [idx3.reshape(NBOXES, PTS_PAD),
         jnp.zeros((padn, PTS_PAD), jnp.int32)]).reshape(
             NBOXES_PAD // 2, 2 * PTS_PAD)
    w2 = jnp.concatenate(
        [w3.reshape(NBOXES, TAPS_PAD), jnp.zeros((padn, TAPS_PAD), jnp.float32)])
    out = _sc_roi()(table, idx2, w2)
    return out.reshape(NB, NBOX, C, S, S)
